# baseline (speedup 1.0000x reference)
import functools

import jax
import jax.numpy as jnp
from jax import lax
from jax.experimental import pallas as pl
from jax.experimental.pallas import tpu as pltpu
from jax.experimental.pallas import tpu_sc as plsc

NC, NS = 2, 16
NW = NC * NS
BATCH = 16384
D = 64
B_PER_W = BATCH // NW


def kernel(inp, embedding, context_embedding):
    idx = inp.astype(jnp.int32)
    mesh = plsc.VectorSubcoreMesh(
        core_axis_name="c", subcore_axis_name="s", num_cores=NC, num_subcores=NS
    )

    @functools.partial(
        pl.kernel,
        out_type=jax.ShapeDtypeStruct((BATCH, 2 * D), jnp.float32),
        mesh=mesh,
        scratch_types=[
            pltpu.VMEM((B_PER_W,), jnp.int32),
            pltpu.VMEM((B_PER_W, D), jnp.float32),
            pltpu.VMEM((B_PER_W, D), jnp.float32),
            pltpu.SemaphoreType.DMA,
            pltpu.SemaphoreType.DMA,
        ],
        compiler_params=pltpu.CompilerParams(use_tc_tiling_on_sc=False),
    )
    def _gather2(idx_hbm, emb_hbm, ctx_hbm, out_hbm,
                 idx_v, rows1_v, rows2_v, sem1, sem2):
        wid = lax.axis_index("s") * NC + lax.axis_index("c")
        base = wid * B_PER_W
        pltpu.sync_copy(idx_hbm.at[pl.ds(base, B_PER_W)], idx_v)
        c1 = pltpu.async_copy(emb_hbm.at[idx_v], rows1_v, sem1)
        c2 = pltpu.async_copy(ctx_hbm.at[idx_v], rows2_v, sem2)
        c1.wait()
        pltpu.sync_copy(rows1_v, out_hbm.at[pl.ds(base, B_PER_W), pl.ds(0, D)])
        c2.wait()
        pltpu.sync_copy(rows2_v, out_hbm.at[pl.ds(base, B_PER_W), pl.ds(D, D)])

    return _gather2(idx, embedding, context_embedding)


# speedup vs baseline: 1.8864x; 1.8864x over previous
import functools

import jax
import jax.numpy as jnp
from jax import lax
from jax.experimental import pallas as pl
from jax.experimental.pallas import tpu as pltpu
from jax.experimental.pallas import tpu_sc as plsc

NC, NS = 2, 16
NW = NC * NS
BATCH = 16384
D = 64
B_PER_W = BATCH // NW
NODE_TILES = 125000
K = 16
CH = B_PER_W // K


def kernel(inp, embedding, context_embedding):
    idx = inp.astype(jnp.int32)
    emb3 = embedding.reshape(NODE_TILES, 8, D)
    ctx3 = context_embedding.reshape(NODE_TILES, 8, D)
    mesh = plsc.VectorSubcoreMesh(
        core_axis_name="c", subcore_axis_name="s", num_cores=NC, num_subcores=NS
    )

    @functools.partial(
        pl.kernel,
        out_type=jax.ShapeDtypeStruct((BATCH, 2 * D), jnp.float32),
        mesh=mesh,
        scratch_types=[
            pltpu.VMEM((B_PER_W,), jnp.int32),
            pltpu.VMEM((K, 8, D), jnp.float32),
            pltpu.VMEM((K, 8, D), jnp.float32),
            pltpu.VMEM((B_PER_W, 2 * D), jnp.float32),
            pltpu.SemaphoreType.DMA,
            pltpu.SemaphoreType.DMA,
        ],
        compiler_params=pltpu.CompilerParams(needs_layout_passes=False),
    )
    def _gather2(idx_hbm, emb_hbm, ctx_hbm, out_hbm,
                 idx_v, stage_e, stage_c, cat_v, sem_e, sem_c):
        wid = lax.axis_index("s") * NC + lax.axis_index("c")
        base = wid * B_PER_W
        pltpu.sync_copy(idx_hbm.at[pl.ds(base, B_PER_W)], idx_v)
        lanes = lax.iota(jnp.int32, K)

        def chunk(c, carry):
            s = idx_v[pl.ds(c * K, K)]
            tvec = lax.shift_right_logical(s, 3)
            rvec = lax.bitwise_and(s, 7)
            copies = []
            for j in range(K):
                tj = lax.reduce_max(
                    jnp.where(lanes == j, tvec, 0), axes=(0,))
                ce = pltpu.make_async_copy(emb_hbm.at[tj], stage_e.at[j], sem_e)
                cc = pltpu.make_async_copy(ctx_hbm.at[tj], stage_c.at[j], sem_c)
                ce.start()
                cc.start()
                copies.append((ce, cc))
            for ce, cc in copies:
                ce.wait()
                cc.wait()
            rows = c * K + lanes
            for q in range(D):
                qv = jnp.full((K,), q, jnp.int32)
                ve = plsc.load_gather(stage_e, [lanes, rvec, qv])
                plsc.store_scatter(cat_v, [rows, qv], ve)
                vc = plsc.load_gather(stage_c, [lanes, rvec, qv])
                plsc.store_scatter(cat_v, [rows, qv + D], vc)
            return carry

        lax.fori_loop(0, CH, chunk, 0)
        pltpu.sync_copy(cat_v, out_hbm.at[pl.ds(base, B_PER_W), :])

    return _gather2(idx, emb3, ctx3)
